# baseline (speedup 1.0000x reference)
import functools

import jax
import jax.numpy as jnp
from jax import lax
from jax.experimental import pallas as pl
from jax.experimental.pallas import tpu as pltpu
from jax.experimental.pallas import tpu_sc as plsc

EMBED_DIM = 16
BATCH = 16384
N_FIELDS = 26
TOTAL = BATCH * N_FIELDS

_INFO = plsc.get_sparse_core_info()
NC, NS, NL = _INFO.num_cores, _INFO.num_subcores, _INFO.num_lanes
NW = NC * NS
TC_PER_F = BATCH // 128
UNITS = N_FIELDS * TC_PER_F
UNITS_PER_W = UNITS // NW
K = 8
CHUNK = K * 128
NCHUNK = UNITS_PER_W // K

_MESH = plsc.VectorSubcoreMesh(core_axis_name="c", subcore_axis_name="s")

VOCAB = 1000000
NBLK = VOCAB // 128
BLK_PER_W = 246
BLK_STRIDE = 245
TAIL = VOCAB - NBLK * 128


@functools.partial(
    pl.kernel,
    mesh=_MESH,
    out_type=jax.ShapeDtypeStruct((VOCAB * EMBED_DIM,), jnp.float32),
    scratch_types=[
        pltpu.VMEM((2, 8, 128), jnp.float32),
        pltpu.VMEM((2, 8, 128), jnp.float32),
        pltpu.VMEM((2048,), jnp.float32),
        pltpu.VMEM((2048,), jnp.float32),
        pltpu.SemaphoreType.DMA,
        pltpu.SemaphoreType.DMA,
        pltpu.SemaphoreType.DMA,
        pltpu.SemaphoreType.DMA,
    ],
    compiler_params=pltpu.CompilerParams(needs_layout_passes=False),
)
def _repack_sc(wt_hbm, tail_hbm, out_hbm, in0, in1, ob0, ob1, si0, si1, so0, so1):
    wid = lax.axis_index("s") * NC + lax.axis_index("c")
    start = wid * BLK_STRIDE
    ins = (in0, in1)
    obs = (ob0, ob1)
    sis = (si0, si1)
    sos = (so0, so1)

    def blk_i0(slot):
        blk = jnp.minimum(start + slot, NBLK - 1)
        return blk * 128

    def start_in(j, b):
        i0 = blk_i0(j)
        pltpu.async_copy(wt_hbm.at[pl.ds(0, 8), pl.ds(i0, 128)], ins[b].at[0], sis[b])
        pltpu.async_copy(wt_hbm.at[pl.ds(8, 8), pl.ds(i0, 128)], ins[b].at[1], sis[b])

    def wait_in(b):
        pltpu.make_async_copy(wt_hbm.at[pl.ds(0, 8), pl.ds(0, 128)], ins[b].at[0], sis[b]).wait()
        pltpu.make_async_copy(wt_hbm.at[pl.ds(0, 8), pl.ds(0, 128)], ins[b].at[1], sis[b]).wait()

    def transpose(b):
        for j in range(128):
            iota16 = lax.iota(jnp.int32, 16)
            vals = plsc.load_gather(
                ins[b], [iota16 >> 3, iota16 & 7, jnp.full((16,), j, jnp.int32)]
            )
            obs[b][pl.ds(j * 16, 16)] = vals

    def start_out(j, b):
        i0 = blk_i0(j)
        pltpu.async_copy(obs[b], out_hbm.at[pl.ds(i0 * EMBED_DIM, 2048)], sos[b])

    def wait_out(b):
        pltpu.make_async_copy(obs[b], out_hbm.at[pl.ds(0, 2048)], sos[b]).wait()

    start_in(0, 0)
    start_in(1, 1)

    @pl.loop(0, BLK_PER_W, step=2)
    def _(g):
        for b in range(2):
            j = g + b

            @pl.when(j >= 2)
            def _():
                wait_out(b)

            wait_in(b)
            transpose(b)
            start_out(j, b)

            @pl.when(j + 2 < BLK_PER_W)
            def _():
                start_in(j + 2, b)

    wait_out(0)
    wait_out(1)

    @pl.when(wid == 0)
    def _():
        pltpu.sync_copy(tail_hbm, ob0.at[pl.ds(0, TAIL * EMBED_DIM)])
        pltpu.sync_copy(ob0.at[pl.ds(0, TAIL * EMBED_DIM)],
                        out_hbm.at[pl.ds(NBLK * 128 * EMBED_DIM, TAIL * EMBED_DIM)])


@functools.partial(
    pl.kernel,
    mesh=_MESH,
    out_type=jax.ShapeDtypeStruct((N_FIELDS, 2, TC_PER_F, 8, 128), jnp.float32),
    scratch_types=[
        pltpu.VMEM((CHUNK,), jnp.int32),
        pltpu.VMEM((CHUNK,), jnp.int32),
        pltpu.VMEM((CHUNK, EMBED_DIM), jnp.float32),
        pltpu.VMEM((CHUNK, EMBED_DIM), jnp.float32),
        pltpu.VMEM((2, K, 8, 128), jnp.float32),
        pltpu.VMEM((2, K, 8, 128), jnp.float32),
        pltpu.SemaphoreType.DMA,
        pltpu.SemaphoreType.DMA,
        pltpu.SemaphoreType.DMA,
    ],
    compiler_params=pltpu.CompilerParams(
        use_tc_tiling_on_sc=False, needs_layout_passes=False
    ),
)
def _gather_sc(table_hbm, idx_hbm, out_hbm,
               idx0, idx1, rows0, rows1, tiles0, tiles1,
               sem_i, sem_g, sem_w):
    wid = lax.axis_index("s") * NC + lax.axis_index("c")
    base = wid * UNITS_PER_W * 128
    idx_b = (idx0, idx1)
    row_b = (rows0, rows1)
    tile_b = (tiles0, tiles1)

    def idx_copy(i):
        off = base + i * CHUNK
        return pltpu.async_copy(idx_hbm.at[pl.ds(off, CHUNK)], idx_b[i % 2], sem_i)

    def gather(i):
        return pltpu.async_copy(table_hbm.at[idx_b[i % 2]], row_b[i % 2], sem_g)

    def transpose(i):
        rows = row_b[i % 2]
        tiles = tile_b[i % 2]

        def body(t, carry):
            iota16 = lax.iota(jnp.int32, 16)
            tr = t >> 6
            k = (t >> 3) & 7
            s = t & 7
            d = tr * 8 + s
            col = jnp.full((16,), d, dtype=jnp.int32)
            for g in range(8):
                row_idx = iota16 + (k * 128 + g * 16)
                vals = plsc.load_gather(rows, [row_idx, col])
                tiles[tr, k, s, pl.ds(g * 16, 16)] = vals
            return carry

        lax.fori_loop(0, 128, body, 0)

    def write(i):
        u0 = wid * UNITS_PER_W + i * K
        f = u0 // TC_PER_F
        tc0 = u0 % TC_PER_F
        tiles = tile_b[i % 2]
        return (
            pltpu.async_copy(tiles.at[0], out_hbm.at[f, 0, pl.ds(tc0, K)], sem_w),
            pltpu.async_copy(tiles.at[1], out_hbm.at[f, 1, pl.ds(tc0, K)], sem_w),
        )

    ic, gc, wc = {}, {}, {}
    ic[0] = idx_copy(0)
    ic[1] = idx_copy(1)
    ic[0].wait()
    gc[0] = gather(0)
    for i in range(NCHUNK):
        gc[i].wait()
        if i + 2 < NCHUNK:
            ic[i + 2] = idx_copy(i + 2)
        if i + 1 < NCHUNK:
            ic[i + 1].wait()
            gc[i + 1] = gather(i + 1)
        if i >= 2:
            for c in wc[i - 2]:
                c.wait()
        transpose(i)
        wc[i] = write(i)
    for j in (NCHUNK - 2, NCHUNK - 1):
        for c in wc[j]:
            c.wait()


def kernel(weight_table, indices):
    idx_flat = indices.T.reshape(-1).astype(jnp.int32)
    tail = weight_table[NBLK * 128:].reshape(-1)
    tbl_lin = _repack_sc(weight_table.T, tail)
    out5d = _gather_sc(tbl_lin.reshape(VOCAB, EMBED_DIM), idx_flat)
    out = out5d.transpose(0, 1, 3, 2, 4).reshape(N_FIELDS, EMBED_DIM, BATCH)
    return out.transpose(2, 0, 1)


# speedup vs baseline: 1.8706x; 1.8706x over previous
import functools

import jax
import jax.numpy as jnp
from jax import lax
from jax.experimental import pallas as pl
from jax.experimental.pallas import tpu as pltpu
from jax.experimental.pallas import tpu_sc as plsc

EMBED_DIM = 16
BATCH = 16384
N_FIELDS = 26
TOTAL = BATCH * N_FIELDS

_INFO = plsc.get_sparse_core_info()
NC, NS, NL = _INFO.num_cores, _INFO.num_subcores, _INFO.num_lanes
NW = NC * NS
TC_PER_F = BATCH // 128
UNITS = N_FIELDS * TC_PER_F
UNITS_PER_W = UNITS // NW
K = 8
CHUNK = K * 128
NCHUNK = UNITS_PER_W // K

_MESH = plsc.VectorSubcoreMesh(core_axis_name="c", subcore_axis_name="s")

VOCAB = 1000000
NBLK = VOCAB // 128
BLK_PER_W = 246
BLK_STRIDE = 245
TAIL = VOCAB - NBLK * 128


@functools.partial(
    pl.kernel,
    mesh=_MESH,
    out_type=jax.ShapeDtypeStruct((VOCAB * EMBED_DIM,), jnp.float32),
    scratch_types=[
        pltpu.VMEM((16, 128), jnp.float32),
        pltpu.VMEM((16, 128), jnp.float32),
        pltpu.VMEM((2048,), jnp.float32),
        pltpu.VMEM((2048,), jnp.float32),
        pltpu.SemaphoreType.DMA,
        pltpu.SemaphoreType.DMA,
        pltpu.SemaphoreType.DMA,
        pltpu.SemaphoreType.DMA,
    ],
    compiler_params=pltpu.CompilerParams(needs_layout_passes=False),
)
def _repack_sc(wt_hbm, tail_hbm, out_hbm, in0, in1, ob0, ob1, si0, si1, so0, so1):
    wid = lax.axis_index("s") * NC + lax.axis_index("c")
    start = wid * BLK_STRIDE
    ins = (in0, in1)
    obs = (ob0, ob1)
    sis = (si0, si1)
    sos = (so0, so1)

    def blk_i0(slot):
        blk = jnp.minimum(start + slot, NBLK - 1)
        return blk * 128

    def start_in(j, b):
        i0 = blk_i0(j)
        pltpu.async_copy(wt_hbm.at[pl.ds(0, 8), pl.ds(i0, 128)],
                         ins[b].at[pl.ds(0, 8)], sis[b])
        pltpu.async_copy(wt_hbm.at[pl.ds(8, 8), pl.ds(i0, 128)],
                         ins[b].at[pl.ds(8, 8)], sis[b])

    def wait_in(b):
        pltpu.make_async_copy(wt_hbm.at[pl.ds(0, 8), pl.ds(0, 128)],
                              ins[b].at[pl.ds(0, 8)], sis[b]).wait()
        pltpu.make_async_copy(wt_hbm.at[pl.ds(0, 8), pl.ds(0, 128)],
                              ins[b].at[pl.ds(8, 8)], sis[b]).wait()

    def transpose(b):
        iota16 = lax.iota(jnp.int32, 16)
        voff = iota16 * EMBED_DIM
        for d in range(EMBED_DIM):
            for g in range(8):
                v = ins[b][d, pl.ds(g * 16, 16)]
                plsc.store_scatter(obs[b], [voff + (g * 16 * EMBED_DIM + d)], v)

    def start_out(j, b):
        i0 = blk_i0(j)
        pltpu.async_copy(obs[b], out_hbm.at[pl.ds(i0 * EMBED_DIM, 2048)], sos[b])

    def wait_out(b):
        pltpu.make_async_copy(obs[b], out_hbm.at[pl.ds(0, 2048)], sos[b]).wait()

    start_in(0, 0)
    start_in(1, 1)

    @pl.loop(0, BLK_PER_W, step=2)
    def _(g):
        for b in range(2):
            j = g + b

            @pl.when(j >= 2)
            def _():
                wait_out(b)

            wait_in(b)
            transpose(b)
            start_out(j, b)

            @pl.when(j + 2 < BLK_PER_W)
            def _():
                start_in(j + 2, b)

    wait_out(0)
    wait_out(1)

    @pl.when(wid == 0)
    def _():
        pltpu.sync_copy(tail_hbm, ob0.at[pl.ds(0, TAIL * EMBED_DIM)])
        pltpu.sync_copy(ob0.at[pl.ds(0, TAIL * EMBED_DIM)],
                        out_hbm.at[pl.ds(NBLK * 128 * EMBED_DIM, TAIL * EMBED_DIM)])


@functools.partial(
    pl.kernel,
    mesh=_MESH,
    out_type=jax.ShapeDtypeStruct((N_FIELDS, 2, TC_PER_F, 8, 128), jnp.float32),
    scratch_types=[
        pltpu.VMEM((CHUNK,), jnp.int32),
        pltpu.VMEM((CHUNK,), jnp.int32),
        pltpu.VMEM((CHUNK, EMBED_DIM), jnp.float32),
        pltpu.VMEM((CHUNK, EMBED_DIM), jnp.float32),
        pltpu.VMEM((2, K, 8, 128), jnp.float32),
        pltpu.VMEM((2, K, 8, 128), jnp.float32),
        pltpu.SemaphoreType.DMA,
        pltpu.SemaphoreType.DMA,
        pltpu.SemaphoreType.DMA,
    ],
    compiler_params=pltpu.CompilerParams(
        use_tc_tiling_on_sc=False, needs_layout_passes=False
    ),
)
def _gather_sc(table_hbm, idx_hbm, out_hbm,
               idx0, idx1, rows0, rows1, tiles0, tiles1,
               sem_i, sem_g, sem_w):
    wid = lax.axis_index("s") * NC + lax.axis_index("c")
    base = wid * UNITS_PER_W * 128
    idx_b = (idx0, idx1)
    row_b = (rows0, rows1)
    tile_b = (tiles0, tiles1)

    def idx_copy(i):
        off = base + i * CHUNK
        return pltpu.async_copy(idx_hbm.at[pl.ds(off, CHUNK)], idx_b[i % 2], sem_i)

    def gather(i):
        return pltpu.async_copy(table_hbm.at[idx_b[i % 2]], row_b[i % 2], sem_g)

    def transpose(i):
        rows = row_b[i % 2]
        tiles = tile_b[i % 2]

        def body(t, carry):
            iota16 = lax.iota(jnp.int32, 16)
            tr = t >> 6
            k = (t >> 3) & 7
            s = t & 7
            d = tr * 8 + s
            col = jnp.full((16,), d, dtype=jnp.int32)
            for g in range(8):
                row_idx = iota16 + (k * 128 + g * 16)
                vals = plsc.load_gather(rows, [row_idx, col])
                tiles[tr, k, s, pl.ds(g * 16, 16)] = vals
            return carry

        lax.fori_loop(0, 128, body, 0)

    def write(i):
        u0 = wid * UNITS_PER_W + i * K
        f = u0 // TC_PER_F
        tc0 = u0 % TC_PER_F
        tiles = tile_b[i % 2]
        return (
            pltpu.async_copy(tiles.at[0], out_hbm.at[f, 0, pl.ds(tc0, K)], sem_w),
            pltpu.async_copy(tiles.at[1], out_hbm.at[f, 1, pl.ds(tc0, K)], sem_w),
        )

    ic, gc, wc = {}, {}, {}
    ic[0] = idx_copy(0)
    ic[1] = idx_copy(1)
    ic[0].wait()
    gc[0] = gather(0)
    for i in range(NCHUNK):
        gc[i].wait()
        if i + 2 < NCHUNK:
            ic[i + 2] = idx_copy(i + 2)
        if i + 1 < NCHUNK:
            ic[i + 1].wait()
            gc[i + 1] = gather(i + 1)
        if i >= 2:
            for c in wc[i - 2]:
                c.wait()
        transpose(i)
        wc[i] = write(i)
    for j in (NCHUNK - 2, NCHUNK - 1):
        for c in wc[j]:
            c.wait()


def kernel(weight_table, indices):
    idx_flat = indices.T.reshape(-1).astype(jnp.int32)
    tail = weight_table[NBLK * 128:].reshape(-1)
    tbl_lin = _repack_sc(weight_table.T, tail)
    out5d = _gather_sc(tbl_lin.reshape(VOCAB, EMBED_DIM), idx_flat)
    out = out5d.transpose(0, 1, 3, 2, 4).reshape(N_FIELDS, EMBED_DIM, BATCH)
    return out.transpose(2, 0, 1)
